# baseline (speedup 1.0000x reference)
import functools

import jax
import jax.numpy as jnp
from jax import lax
from jax.experimental import pallas as pl
from jax.experimental.pallas import tpu as pltpu
from jax.experimental.pallas import tpu_sc as plsc

N = 10000
E = 160000
D = 300
DH = 150
DP = 160
B = 256
NC, NS = 2, 16
NP = 10240
KW = 80
CW = 10
EPT = E // NS
DEG_W = 1000
EPW = E // (NC * NS)

@functools.cache
def _sc_degree_kernel():
    mesh = plsc.VectorSubcoreMesh(core_axis_name="c", subcore_axis_name="s",
                                  num_cores=NC, num_subcores=NS)
    return pl.kernel(
        _sc_degree_body,
        out_type=jax.ShapeDtypeStruct((NC, NP), jnp.float32),
        mesh=mesh,
        compiler_params=pltpu.CompilerParams(use_tc_tiling_on_sc=False),
        scratch_types=[
            pltpu.VMEM((1008,), jnp.float32),
            pltpu.VMEM((DEG_W,), jnp.int32),
            pltpu.VMEM((NP // NS,), jnp.float32),
            pltpu.VMEM_SHARED((NP,), jnp.float32),
        ],
    )


def _sc_degree_body(dst_hbm, deg_out, ones_v, idx_v, zero_v, deg_sh):
    c = lax.axis_index("c")
    s = lax.axis_index("s")
    zc = jnp.zeros((16,), jnp.float32)
    oc = jnp.ones((16,), jnp.float32)

    def fill_z(i, carry):
        zero_v[pl.ds(i * 16, 16)] = zc
        return carry

    lax.fori_loop(0, (NP // NS) // 16, fill_z, 0)

    def fill_o(i, carry):
        ones_v[pl.ds(i * 16, 16)] = oc
        return carry

    lax.fori_loop(0, 1008 // 16, fill_o, 0)

    pltpu.sync_copy(zero_v, deg_sh.at[pl.ds(s * (NP // NS), NP // NS)])
    plsc.subcore_barrier()

    def win(w, carry):
        base = pl.multiple_of(c * (E // NC) + s * EPW + w * DEG_W, 8)
        pltpu.sync_copy(dst_hbm.at[pl.ds(base, DEG_W)], idx_v)
        pltpu.sync_copy(ones_v.at[pl.ds(0, DEG_W)], deg_sh.at[idx_v], add=True)
        return carry

    lax.fori_loop(0, EPW // DEG_W, win, 0)
    plsc.subcore_barrier()
    pltpu.sync_copy(deg_sh.at[pl.ds(s * (NP // NS), NP // NS)],
                    deg_out.at[c, pl.ds(s * (NP // NS), NP // NS)])


@functools.cache
def _sc_spmm_kernel():
    mesh = plsc.VectorSubcoreMesh(core_axis_name="c", subcore_axis_name="s",
                                  num_cores=NC, num_subcores=NS)
    return pl.kernel(
        _sc_spmm_body,
        out_type=jax.ShapeDtypeStruct((NC, NP, DP), jnp.float32),
        mesh=mesh,
        compiler_params=pltpu.CompilerParams(use_tc_tiling_on_sc=False),
        scratch_types=[
            pltpu.VMEM((CW * KW,), jnp.int32),
            pltpu.VMEM((CW, KW), jnp.int32),
            pltpu.VMEM((KW, DP), jnp.float32),
            pltpu.VMEM((KW, DP), jnp.float32),
            pltpu.SemaphoreType.DMA,
            pltpu.SemaphoreType.DMA,
            pltpu.SemaphoreType.DMA,
            pltpu.SemaphoreType.DMA,
            pltpu.VMEM_SHARED((NP, DP), jnp.float32),
        ],
    )


def _sc_spmm_body(hn_hbm, src_hbm, dst2_hbm, agg_out,
                  gidx_v, dstw_v, rows0, rows1, gsem0, gsem1, ssem0, ssem1,
                  agg_sh):
    c = lax.axis_index("c")
    s = lax.axis_index("s")
    zc = jnp.zeros((16,), jnp.float32)

    def fill_z(i, carry):
        rows0[i // (DP // 16), pl.ds((i % (DP // 16)) * 16, 16)] = zc
        return carry

    lax.fori_loop(0, KW * (DP // 16), fill_z, 0)

    def zrows(j, carry):
        pltpu.sync_copy(rows0, agg_sh.at[pl.ds(s * 640 + j * KW, KW)])
        return carry

    lax.fori_loop(0, 640 // KW, zrows, 0)
    plsc.subcore_barrier()

    coff = c * N
    tile_base = s * EPT
    tile_wbase = s * (EPT // KW)
    rows = (rows0, rows1)
    gsem = (gsem0, gsem1)
    ssem = (ssem0, ssem1)

    def load_chunk(ch):
        base = pl.multiple_of(tile_base + ch * (CW * KW), 8)
        pltpu.sync_copy(src_hbm.at[pl.ds(base, CW * KW)], gidx_v)
        pltpu.sync_copy(dst2_hbm.at[pl.ds(tile_wbase + ch * CW, CW)], dstw_v)

        def addoff(i, carry2):
            gidx_v[pl.ds(i * 16, 16)] = gidx_v[pl.ds(i * 16, 16)] + coff
            return carry2

        lax.fori_loop(0, (CW * KW) // 16, addoff, 0)

    def issue_gather(k, p):
        pltpu.async_copy(hn_hbm.at[gidx_v.at[pl.ds(k * KW, KW)]],
                         rows[p], gsem[p])

    def wait_gather(p):
        pltpu.make_async_copy(hn_hbm.at[gidx_v.at[pl.ds(0, KW)]],
                              rows[p], gsem[p]).wait()

    def issue_scatter(k, p):
        pltpu.async_copy(rows[p], agg_sh.at[dstw_v.at[k]], ssem[p], add=True)

    def wait_scatter(p):
        pltpu.make_async_copy(rows[p], agg_sh.at[dstw_v.at[0]], ssem[p]).wait()

    NWIN = EPT // KW
    NCH = NWIN // CW

    load_chunk(0)
    issue_gather(0, 0)

    def chunk_body(ch, carry):
        for k in range(CW):
            p = k % 2
            if k > 0:
                wait_scatter(1 - p)
            wait_gather(p)
            if k < CW - 1:
                issue_gather(k + 1, 1 - p)
            issue_scatter(k, p)
        wait_scatter((CW - 1) % 2)
        load_chunk(ch + 1)
        issue_gather(0, 0)
        return carry

    lax.fori_loop(0, NCH, chunk_body, 0)

    TAIL = NWIN - NCH * CW

    for k in range(TAIL):
        p = k % 2
        if k > 0:
            wait_scatter(1 - p)
        wait_gather(p)
        if k < TAIL - 1:
            issue_gather(k + 1, 1 - p)
        issue_scatter(k, p)
    wait_scatter((TAIL - 1) % 2)
    plsc.subcore_barrier()

    def wout(j, carry):
        r0 = s * 640 + j * 128
        pltpu.sync_copy(agg_sh.at[pl.ds(r0, 128)], agg_out.at[c, pl.ds(r0, 128)])
        return carry

    lax.fori_loop(0, 5, wout, 0)


RB = 1000


def _split_pad_stack(hn):
    a = jnp.pad(hn[:, :DH], ((0, 0), (0, DP - DH)))
    b = jnp.pad(hn[:, DH:], ((0, 0), (0, DP - DH)))
    return jnp.stack([a, b], axis=0)


def _tc_pre_body(x_ref, w_ref, b_ref, degp_ref, hn_ref, selfb_ref, inv_ref):
    deg = degp_ref[0] + degp_ref[1] + 1.0
    inv = lax.rsqrt(deg)
    h = jnp.dot(x_ref[...], w_ref[...], preferred_element_type=jnp.float32)
    hn_ref[...] = _split_pad_stack(h * inv)
    selfb_ref[...] = h * (inv * inv) + b_ref[...]
    inv_ref[...] = inv


def _tc_pre(x, w, bias, degp):
    return pl.pallas_call(
        _tc_pre_body,
        grid=(N // RB,),
        in_specs=[
            pl.BlockSpec((RB, D), lambda i: (i, 0)),
            pl.BlockSpec((D, D), lambda i: (0, 0)),
            pl.BlockSpec((1, D), lambda i: (0, 0)),
            pl.BlockSpec((NC, RB, 1), lambda i: (0, i, 0)),
        ],
        out_specs=[
            pl.BlockSpec((NC, RB, DP), lambda i: (0, i, 0)),
            pl.BlockSpec((RB, D), lambda i: (i, 0)),
            pl.BlockSpec((RB, 1), lambda i: (i, 0)),
        ],
        out_shape=[
            jax.ShapeDtypeStruct((NC, N, DP), jnp.float32),
            jax.ShapeDtypeStruct((N, D), jnp.float32),
            jax.ShapeDtypeStruct((N, 1), jnp.float32),
        ],
    )(x, w, bias, degp)


def _gcn_out(agg_ref, selfb_ref, inv_ref):
    aggcat = jnp.concatenate([agg_ref[0, :, :DH], agg_ref[1, :, :DH]], axis=1)
    return jnp.maximum(inv_ref[...] * aggcat + selfb_ref[...], 0.0)


def _tc_mid_body(agg_ref, selfb_ref, inv_ref, w_ref, b_ref, hn_ref, selfb2_ref):
    g = _gcn_out(agg_ref, selfb_ref, inv_ref)
    h = jnp.dot(g, w_ref[...], preferred_element_type=jnp.float32)
    inv = inv_ref[...]
    hn_ref[...] = _split_pad_stack(h * inv)
    selfb2_ref[...] = h * (inv * inv) + b_ref[...]


def _tc_mid(agg, selfb, inv, w, bias):
    return pl.pallas_call(
        _tc_mid_body,
        grid=(N // RB,),
        in_specs=[
            pl.BlockSpec((NC, RB, DP), lambda i: (0, i, 0)),
            pl.BlockSpec((RB, D), lambda i: (i, 0)),
            pl.BlockSpec((RB, 1), lambda i: (i, 0)),
            pl.BlockSpec((D, D), lambda i: (0, 0)),
            pl.BlockSpec((1, D), lambda i: (0, 0)),
        ],
        out_specs=[
            pl.BlockSpec((NC, RB, DP), lambda i: (0, i, 0)),
            pl.BlockSpec((RB, D), lambda i: (i, 0)),
        ],
        out_shape=[
            jax.ShapeDtypeStruct((NC, N, DP), jnp.float32),
            jax.ShapeDtypeStruct((N, D), jnp.float32),
        ],
    )(agg, selfb, inv, w, bias)


def _tc_final_body(agg_ref, selfb_ref, inv_ref, mol_ref, w1a_ref, w1b_ref,
                   b1_ref, w2_ref, b2_ref, w3_ref, b3_ref, w4_ref, b4_ref,
                   out_ref, acc_ref):
    i = pl.program_id(0)
    g = _gcn_out(agg_ref, selfb_ref, inv_ref)
    part = jnp.sum(g, axis=0, keepdims=True)

    @pl.when(i == 0)
    def _():
        acc_ref[...] = part

    @pl.when(i > 0)
    def _():
        acc_ref[...] = acc_ref[...] + part

    @pl.when(i == pl.num_programs(0) - 1)
    def _():
        pooled = acc_ref[...] * (1.0 / N)
        t = (jnp.dot(mol_ref[...], w1a_ref[...], preferred_element_type=jnp.float32)
             + jnp.dot(pooled, w1b_ref[...], preferred_element_type=jnp.float32)
             + b1_ref[...])
        t = jnp.maximum(t, 0.0)
        t = jnp.maximum(jnp.dot(t, w2_ref[...], preferred_element_type=jnp.float32) + b2_ref[...], 0.0)
        t = jnp.maximum(jnp.dot(t, w3_ref[...], preferred_element_type=jnp.float32) + b3_ref[...], 0.0)
        out_ref[...] = jnp.dot(t, w4_ref[...], preferred_element_type=jnp.float32) + b4_ref[...]


def _tc_final(agg, selfb, inv, mol_emb, w1a, w1b, b1, w2, b2, w3, b3, w4, b4):
    full = lambda shape: pl.BlockSpec(shape, lambda i: tuple(0 for _ in shape))
    h = w2.shape[0]
    o = w4.shape[1]
    return pl.pallas_call(
        _tc_final_body,
        grid=(N // RB,),
        in_specs=[
            pl.BlockSpec((NC, RB, DP), lambda i: (0, i, 0)),
            pl.BlockSpec((RB, D), lambda i: (i, 0)),
            pl.BlockSpec((RB, 1), lambda i: (i, 0)),
            full((B, D)),
            full((D, h)),
            full((D, h)),
            full((1, h)),
            full((h, h)),
            full((1, h)),
            full((h, h)),
            full((1, h)),
            full((h, o)),
            full((1, o)),
        ],
        out_specs=full((B, o)),
        out_shape=jax.ShapeDtypeStruct((B, o), jnp.float32),
        scratch_shapes=[pltpu.VMEM((1, D), jnp.float32)],
    )(agg, selfb, inv, mol_emb, w1a, w1b, b1, w2, b2, w3, b3, w4, b4)


def kernel(mol_emb, gene_ids, edge_index, emb_table, W_g1, b_g1, W_g2, b_g2,
           W1, b1, W2, b2, W3, b3, W4, b4):
    src = edge_index[0]
    dst = edge_index[1]
    degp = _sc_degree_kernel()(dst)
    degp_n = degp[:, :N].reshape(NC, N, 1)
    hn1, selfb1, inv = _tc_pre(emb_table, W_g1, b_g1.reshape(1, D), degp_n)
    src_p = jnp.pad(src, (0, CW * KW))
    dst2 = jnp.pad(dst.reshape(E // KW, KW), ((0, CW), (0, 0)))
    agg1 = _sc_spmm_kernel()(hn1.reshape(NC * N, DP), src_p, dst2)
    hn2, selfb2 = _tc_mid(agg1, selfb1, inv, W_g2, b_g2.reshape(1, D))
    agg2 = _sc_spmm_kernel()(hn2.reshape(NC * N, DP), src_p, dst2)
    return _tc_final(agg2, selfb2, inv, mol_emb,
                     W1[:D], W1[D:], b1.reshape(1, -1),
                     W2, b2.reshape(1, -1), W3, b3.reshape(1, -1),
                     W4, b4.reshape(1, -1))


# speedup vs baseline: 1.0805x; 1.0805x over previous
import functools

import jax
import jax.numpy as jnp
from jax import lax
from jax.experimental import pallas as pl
from jax.experimental.pallas import tpu as pltpu
from jax.experimental.pallas import tpu_sc as plsc

N = 10000
E = 160000
D = 300
DH = 150
DP = 160
B = 256
NC, NS = 2, 16
NP = 10240
KW = 80
EPT = E // NS
DEG_W = 1000
EPW = E // (NC * NS)

@functools.cache
def _sc_degree_kernel():
    mesh = plsc.VectorSubcoreMesh(core_axis_name="c", subcore_axis_name="s",
                                  num_cores=NC, num_subcores=NS)
    return pl.kernel(
        _sc_degree_body,
        out_type=jax.ShapeDtypeStruct((NC, NP), jnp.float32),
        mesh=mesh,
        compiler_params=pltpu.CompilerParams(use_tc_tiling_on_sc=False),
        scratch_types=[
            pltpu.VMEM((1008,), jnp.float32),
            pltpu.VMEM((DEG_W,), jnp.int32),
            pltpu.VMEM((NP // NS,), jnp.float32),
            pltpu.VMEM_SHARED((NP,), jnp.float32),
        ],
    )


def _sc_degree_body(dst_hbm, deg_out, ones_v, idx_v, zero_v, deg_sh):
    c = lax.axis_index("c")
    s = lax.axis_index("s")
    zc = jnp.zeros((16,), jnp.float32)
    oc = jnp.ones((16,), jnp.float32)

    def fill_z(i, carry):
        zero_v[pl.ds(i * 16, 16)] = zc
        return carry

    lax.fori_loop(0, (NP // NS) // 16, fill_z, 0)

    def fill_o(i, carry):
        ones_v[pl.ds(i * 16, 16)] = oc
        return carry

    lax.fori_loop(0, 1008 // 16, fill_o, 0)

    pltpu.sync_copy(zero_v, deg_sh.at[pl.ds(s * (NP // NS), NP // NS)])
    plsc.subcore_barrier()

    def win(w, carry):
        base = pl.multiple_of(c * (E // NC) + s * EPW + w * DEG_W, 8)
        pltpu.sync_copy(dst_hbm.at[pl.ds(base, DEG_W)], idx_v)
        pltpu.sync_copy(ones_v.at[pl.ds(0, DEG_W)], deg_sh.at[idx_v], add=True)
        return carry

    lax.fori_loop(0, EPW // DEG_W, win, 0)
    plsc.subcore_barrier()
    pltpu.sync_copy(deg_sh.at[pl.ds(s * (NP // NS), NP // NS)],
                    deg_out.at[c, pl.ds(s * (NP // NS), NP // NS)])


@functools.cache
def _sc_spmm_kernel():
    mesh = plsc.VectorSubcoreMesh(core_axis_name="c", subcore_axis_name="s",
                                  num_cores=NC, num_subcores=NS)
    return pl.kernel(
        _sc_spmm_body,
        out_type=jax.ShapeDtypeStruct((NC, NP, DP), jnp.float32),
        mesh=mesh,
        compiler_params=pltpu.CompilerParams(use_tc_tiling_on_sc=False),
        scratch_types=[
            pltpu.VMEM((KW,), jnp.int32),
            pltpu.VMEM((KW,), jnp.int32),
            pltpu.VMEM((KW,), jnp.int32),
            pltpu.VMEM((KW,), jnp.int32),
            pltpu.VMEM((KW,), jnp.int32),
            pltpu.VMEM((KW,), jnp.int32),
            pltpu.VMEM((KW, DP), jnp.float32),
            pltpu.VMEM((KW, DP), jnp.float32),
            pltpu.SemaphoreType.DMA,
            pltpu.SemaphoreType.DMA,
            pltpu.SemaphoreType.DMA,
            pltpu.SemaphoreType.DMA,
            pltpu.SemaphoreType.DMA,
            pltpu.SemaphoreType.DMA,
            pltpu.VMEM_SHARED((NP, DP), jnp.float32),
        ],
    )


def _sc_spmm_body(hn_hbm, src_hbm, dst_hbm, agg_out,
                  srcw0, srcw1, gidx0, gidx1, dstw0, dstw1,
                  rows0, rows1, gsem0, gsem1, ssem0, ssem1, isem0, isem1,
                  agg_sh):
    c = lax.axis_index("c")
    s = lax.axis_index("s")
    zc = jnp.zeros((16,), jnp.float32)

    def fill_z(i, carry):
        rows0[i // (DP // 16), pl.ds((i % (DP // 16)) * 16, 16)] = zc
        return carry

    lax.fori_loop(0, KW * (DP // 16), fill_z, 0)

    def zrows(j, carry):
        pltpu.sync_copy(rows0, agg_sh.at[pl.ds(s * 640 + j * KW, KW)])
        return carry

    lax.fori_loop(0, 640 // KW, zrows, 0)
    plsc.subcore_barrier()

    coff = c * N
    tile_base = s * EPT
    bufs = ((srcw0, gidx0, dstw0, rows0, gsem0, ssem0, isem0),
            (srcw1, gidx1, dstw1, rows1, gsem1, ssem1, isem1))

    def issue_idx(w, p):
        srcw, _, dstw, _, _, _, isem = bufs[p]
        base = pl.multiple_of(tile_base + w * KW, 8)
        pltpu.async_copy(src_hbm.at[pl.ds(base, KW)], srcw, isem)
        pltpu.async_copy(dst_hbm.at[pl.ds(base, KW)], dstw, isem)

    def wait_idx_make_gidx(p):
        srcw, gidx, dstw, _, _, _, isem = bufs[p]
        pltpu.make_async_copy(src_hbm.at[pl.ds(0, KW)], srcw, isem).wait()
        pltpu.make_async_copy(dst_hbm.at[pl.ds(0, KW)], dstw, isem).wait()

        def addoff(i, carry2):
            gidx[pl.ds(i * 16, 16)] = srcw[pl.ds(i * 16, 16)] + coff
            return carry2

        lax.fori_loop(0, KW // 16, addoff, 0)

    def issue_gather(p):
        _, gidx, _, rows, gsem, _, _ = bufs[p]
        pltpu.async_copy(hn_hbm.at[gidx], rows, gsem)

    def wait_gather(p):
        _, gidx, _, rows, gsem, _, _ = bufs[p]
        pltpu.make_async_copy(hn_hbm.at[gidx], rows, gsem).wait()

    def issue_scatter(p):
        _, _, dstw, rows, _, ssem, _ = bufs[p]
        pltpu.async_copy(rows, agg_sh.at[dstw], ssem, add=True)

    def wait_scatter(p):
        _, _, dstw, rows, _, ssem, _ = bufs[p]
        pltpu.make_async_copy(rows, agg_sh.at[dstw], ssem).wait()

    def half(w, p):
        wait_scatter(1 - p)
        issue_idx(w + 1, 1 - p)
        wait_gather(p)
        wait_idx_make_gidx(1 - p)
        issue_gather(1 - p)
        issue_scatter(p)

    issue_idx(0, 0)
    wait_idx_make_gidx(0)
    issue_gather(0)
    issue_idx(1, 1)
    wait_gather(0)
    wait_idx_make_gidx(1)
    issue_gather(1)
    issue_scatter(0)

    def pair(g, carry):
        half(2 * g + 1, 1)
        half(2 * g + 2, 0)
        return carry

    NW = EPT // KW
    lax.fori_loop(0, (NW - 3) // 2, pair, 0)
    half(NW - 2, 1)
    wait_scatter(1)
    wait_gather(0)
    issue_scatter(0)
    wait_scatter(0)
    plsc.subcore_barrier()

    def wout(j, carry):
        r0 = s * 640 + j * 128
        pltpu.sync_copy(agg_sh.at[pl.ds(r0, 128)], agg_out.at[c, pl.ds(r0, 128)])
        return carry

    lax.fori_loop(0, 5, wout, 0)


RB = 1000


def _split_pad_stack(hn):
    a = jnp.pad(hn[:, :DH], ((0, 0), (0, DP - DH)))
    b = jnp.pad(hn[:, DH:], ((0, 0), (0, DP - DH)))
    return jnp.stack([a, b], axis=0)


def _tc_pre_body(x_ref, w_ref, b_ref, degp_ref, hn_ref, selfb_ref, inv_ref):
    deg = degp_ref[0] + degp_ref[1] + 1.0
    inv = lax.rsqrt(deg)
    h = jnp.dot(x_ref[...], w_ref[...], preferred_element_type=jnp.float32)
    hn_ref[...] = _split_pad_stack(h * inv)
    selfb_ref[...] = h * (inv * inv) + b_ref[...]
    inv_ref[...] = inv


def _tc_pre(x, w, bias, degp):
    return pl.pallas_call(
        _tc_pre_body,
        grid=(N // RB,),
        in_specs=[
            pl.BlockSpec((RB, D), lambda i: (i, 0)),
            pl.BlockSpec((D, D), lambda i: (0, 0)),
            pl.BlockSpec((1, D), lambda i: (0, 0)),
            pl.BlockSpec((NC, RB, 1), lambda i: (0, i, 0)),
        ],
        out_specs=[
            pl.BlockSpec((NC, RB, DP), lambda i: (0, i, 0)),
            pl.BlockSpec((RB, D), lambda i: (i, 0)),
            pl.BlockSpec((RB, 1), lambda i: (i, 0)),
        ],
        out_shape=[
            jax.ShapeDtypeStruct((NC, N, DP), jnp.float32),
            jax.ShapeDtypeStruct((N, D), jnp.float32),
            jax.ShapeDtypeStruct((N, 1), jnp.float32),
        ],
    )(x, w, bias, degp)


def _gcn_out(agg_ref, selfb_ref, inv_ref):
    aggcat = jnp.concatenate([agg_ref[0, :, :DH], agg_ref[1, :, :DH]], axis=1)
    return jnp.maximum(inv_ref[...] * aggcat + selfb_ref[...], 0.0)


def _tc_mid_body(agg_ref, selfb_ref, inv_ref, w_ref, b_ref, hn_ref, selfb2_ref):
    g = _gcn_out(agg_ref, selfb_ref, inv_ref)
    h = jnp.dot(g, w_ref[...], preferred_element_type=jnp.float32)
    inv = inv_ref[...]
    hn_ref[...] = _split_pad_stack(h * inv)
    selfb2_ref[...] = h * (inv * inv) + b_ref[...]


def _tc_mid(agg, selfb, inv, w, bias):
    return pl.pallas_call(
        _tc_mid_body,
        grid=(N // RB,),
        in_specs=[
            pl.BlockSpec((NC, RB, DP), lambda i: (0, i, 0)),
            pl.BlockSpec((RB, D), lambda i: (i, 0)),
            pl.BlockSpec((RB, 1), lambda i: (i, 0)),
            pl.BlockSpec((D, D), lambda i: (0, 0)),
            pl.BlockSpec((1, D), lambda i: (0, 0)),
        ],
        out_specs=[
            pl.BlockSpec((NC, RB, DP), lambda i: (0, i, 0)),
            pl.BlockSpec((RB, D), lambda i: (i, 0)),
        ],
        out_shape=[
            jax.ShapeDtypeStruct((NC, N, DP), jnp.float32),
            jax.ShapeDtypeStruct((N, D), jnp.float32),
        ],
    )(agg, selfb, inv, w, bias)


def _tc_final_body(agg_ref, selfb_ref, inv_ref, mol_ref, w1a_ref, w1b_ref,
                   b1_ref, w2_ref, b2_ref, w3_ref, b3_ref, w4_ref, b4_ref,
                   out_ref, acc_ref):
    i = pl.program_id(0)
    g = _gcn_out(agg_ref, selfb_ref, inv_ref)
    part = jnp.sum(g, axis=0, keepdims=True)

    @pl.when(i == 0)
    def _():
        acc_ref[...] = part

    @pl.when(i > 0)
    def _():
        acc_ref[...] = acc_ref[...] + part

    @pl.when(i == pl.num_programs(0) - 1)
    def _():
        pooled = acc_ref[...] * (1.0 / N)
        t = (jnp.dot(mol_ref[...], w1a_ref[...], preferred_element_type=jnp.float32)
             + jnp.dot(pooled, w1b_ref[...], preferred_element_type=jnp.float32)
             + b1_ref[...])
        t = jnp.maximum(t, 0.0)
        t = jnp.maximum(jnp.dot(t, w2_ref[...], preferred_element_type=jnp.float32) + b2_ref[...], 0.0)
        t = jnp.maximum(jnp.dot(t, w3_ref[...], preferred_element_type=jnp.float32) + b3_ref[...], 0.0)
        out_ref[...] = jnp.dot(t, w4_ref[...], preferred_element_type=jnp.float32) + b4_ref[...]


def _tc_final(agg, selfb, inv, mol_emb, w1a, w1b, b1, w2, b2, w3, b3, w4, b4):
    full = lambda shape: pl.BlockSpec(shape, lambda i: tuple(0 for _ in shape))
    h = w2.shape[0]
    o = w4.shape[1]
    return pl.pallas_call(
        _tc_final_body,
        grid=(N // RB,),
        in_specs=[
            pl.BlockSpec((NC, RB, DP), lambda i: (0, i, 0)),
            pl.BlockSpec((RB, D), lambda i: (i, 0)),
            pl.BlockSpec((RB, 1), lambda i: (i, 0)),
            full((B, D)),
            full((D, h)),
            full((D, h)),
            full((1, h)),
            full((h, h)),
            full((1, h)),
            full((h, h)),
            full((1, h)),
            full((h, o)),
            full((1, o)),
        ],
        out_specs=full((B, o)),
        out_shape=jax.ShapeDtypeStruct((B, o), jnp.float32),
        scratch_shapes=[pltpu.VMEM((1, D), jnp.float32)],
    )(agg, selfb, inv, mol_emb, w1a, w1b, b1, w2, b2, w3, b3, w4, b4)


def kernel(mol_emb, gene_ids, edge_index, emb_table, W_g1, b_g1, W_g2, b_g2,
           W1, b1, W2, b2, W3, b3, W4, b4):
    src = edge_index[0]
    dst = edge_index[1]
    degp = _sc_degree_kernel()(dst)
    degp_n = degp[:, :N].reshape(NC, N, 1)
    hn1, selfb1, inv = _tc_pre(emb_table, W_g1, b_g1.reshape(1, D), degp_n)
    agg1 = _sc_spmm_kernel()(hn1.reshape(NC * N, DP), src, dst)
    hn2, selfb2 = _tc_mid(agg1, selfb1, inv, W_g2, b_g2.reshape(1, D))
    agg2 = _sc_spmm_kernel()(hn2.reshape(NC * N, DP), src, dst)
    return _tc_final(agg2, selfb2, inv, mol_emb,
                     W1[:D], W1[D:], b1.reshape(1, -1),
                     W2, b2.reshape(1, -1), W3, b3.reshape(1, -1),
                     W4, b4.reshape(1, -1))


# speedup vs baseline: 1.0865x; 1.0056x over previous
import functools

import jax
import jax.numpy as jnp
from jax import lax
from jax.experimental import pallas as pl
from jax.experimental.pallas import tpu as pltpu
from jax.experimental.pallas import tpu_sc as plsc

N = 10000
E = 160000
D = 300
DH = 150
DP = 160
B = 256
NC, NS = 2, 16
NP = 10240
KW = 80
EPT = E // NS
DEG_W = 1000
EPW = E // (NC * NS)

@functools.cache
def _sc_degree_kernel():
    mesh = plsc.VectorSubcoreMesh(core_axis_name="c", subcore_axis_name="s",
                                  num_cores=NC, num_subcores=NS)
    return pl.kernel(
        _sc_degree_body,
        out_type=jax.ShapeDtypeStruct((NC, NP), jnp.float32),
        mesh=mesh,
        compiler_params=pltpu.CompilerParams(use_tc_tiling_on_sc=False),
        scratch_types=[
            pltpu.VMEM((1008,), jnp.float32),
            pltpu.VMEM((DEG_W,), jnp.int32),
            pltpu.VMEM((NP // NS,), jnp.float32),
            pltpu.VMEM_SHARED((NP,), jnp.float32),
        ],
    )


def _sc_degree_body(dst_hbm, deg_out, ones_v, idx_v, zero_v, deg_sh):
    c = lax.axis_index("c")
    s = lax.axis_index("s")
    zc = jnp.zeros((16,), jnp.float32)
    oc = jnp.ones((16,), jnp.float32)

    def fill_z(i, carry):
        zero_v[pl.ds(i * 16, 16)] = zc
        return carry

    lax.fori_loop(0, (NP // NS) // 16, fill_z, 0)

    def fill_o(i, carry):
        ones_v[pl.ds(i * 16, 16)] = oc
        return carry

    lax.fori_loop(0, 1008 // 16, fill_o, 0)

    pltpu.sync_copy(zero_v, deg_sh.at[pl.ds(s * (NP // NS), NP // NS)])
    plsc.subcore_barrier()

    def win(w, carry):
        base = pl.multiple_of(c * (E // NC) + s * EPW + w * DEG_W, 8)
        pltpu.sync_copy(dst_hbm.at[pl.ds(base, DEG_W)], idx_v)
        pltpu.sync_copy(ones_v.at[pl.ds(0, DEG_W)], deg_sh.at[idx_v], add=True)
        return carry

    lax.fori_loop(0, EPW // DEG_W, win, 0)
    plsc.subcore_barrier()
    pltpu.sync_copy(deg_sh.at[pl.ds(s * (NP // NS), NP // NS)],
                    deg_out.at[c, pl.ds(s * (NP // NS), NP // NS)])


@functools.cache
def _sc_spmm_kernel():
    mesh = plsc.VectorSubcoreMesh(core_axis_name="c", subcore_axis_name="s",
                                  num_cores=NC, num_subcores=NS)
    return pl.kernel(
        _sc_spmm_body,
        out_type=jax.ShapeDtypeStruct((NC, NP, DP), jnp.float32),
        mesh=mesh,
        compiler_params=pltpu.CompilerParams(use_tc_tiling_on_sc=False),
        scratch_types=[
            pltpu.VMEM((KW,), jnp.int32),
            pltpu.VMEM((KW,), jnp.int32),
            pltpu.VMEM((KW,), jnp.int32),
            pltpu.VMEM((KW,), jnp.int32),
            pltpu.VMEM((KW,), jnp.int32),
            pltpu.VMEM((KW,), jnp.int32),
            pltpu.VMEM((KW, DP), jnp.float32),
            pltpu.VMEM((KW, DP), jnp.float32),
            pltpu.SemaphoreType.DMA,
            pltpu.SemaphoreType.DMA,
            pltpu.SemaphoreType.DMA,
            pltpu.SemaphoreType.DMA,
            pltpu.SemaphoreType.DMA,
            pltpu.SemaphoreType.DMA,
            pltpu.VMEM_SHARED((NP, DP), jnp.float32),
        ],
    )


def _sc_spmm_body(hn_hbm, src_hbm, dst_hbm, agg_out,
                  srcw0, srcw1, gidx0, gidx1, dstw0, dstw1,
                  rows0, rows1, gsem0, gsem1, ssem0, ssem1, isem0, isem1,
                  agg_sh):
    c = lax.axis_index("c")
    s = lax.axis_index("s")
    zc = jnp.zeros((16,), jnp.float32)

    def fill_z(i, carry):
        rows0[i // (DP // 16), pl.ds((i % (DP // 16)) * 16, 16)] = zc
        return carry

    lax.fori_loop(0, KW * (DP // 16), fill_z, 0)

    def zrows(j, carry):
        pltpu.sync_copy(rows0, agg_sh.at[pl.ds(s * 640 + j * KW, KW)])
        return carry

    lax.fori_loop(0, 640 // KW, zrows, 0)
    plsc.subcore_barrier()

    coff = c * N
    tile_base = s * EPT
    bufs = ((srcw0, gidx0, dstw0, rows0, gsem0, ssem0, isem0),
            (srcw1, gidx1, dstw1, rows1, gsem1, ssem1, isem1))

    def issue_idx(w, p):
        srcw, _, dstw, _, _, _, isem = bufs[p]
        base = pl.multiple_of(tile_base + w * KW, 8)
        pltpu.async_copy(src_hbm.at[pl.ds(base, KW)], srcw, isem)
        pltpu.async_copy(dst_hbm.at[pl.ds(base, KW)], dstw, isem)

    def wait_idx_make_gidx(p):
        srcw, gidx, dstw, _, _, _, isem = bufs[p]
        pltpu.make_async_copy(src_hbm.at[pl.ds(0, KW)], srcw, isem).wait()
        pltpu.make_async_copy(dst_hbm.at[pl.ds(0, KW)], dstw, isem).wait()

        def addoff(i, carry2):
            gidx[pl.ds(i * 16, 16)] = srcw[pl.ds(i * 16, 16)] + coff
            return carry2

        lax.fori_loop(0, KW // 16, addoff, 0)

    def issue_gather(p):
        _, gidx, _, rows, gsem, _, _ = bufs[p]
        pltpu.async_copy(hn_hbm.at[gidx], rows, gsem)

    def wait_gather(p):
        _, gidx, _, rows, gsem, _, _ = bufs[p]
        pltpu.make_async_copy(hn_hbm.at[gidx], rows, gsem).wait()

    def issue_scatter(p):
        _, _, dstw, rows, _, ssem, _ = bufs[p]
        pltpu.async_copy(rows, agg_sh.at[dstw], ssem, add=True)

    def wait_scatter(p):
        _, _, dstw, rows, _, ssem, _ = bufs[p]
        pltpu.make_async_copy(rows, agg_sh.at[dstw], ssem).wait()

    def half(w, p):
        wait_scatter(1 - p)
        issue_idx(w + 1, 1 - p)
        wait_gather(p)
        wait_idx_make_gidx(1 - p)
        issue_gather(1 - p)
        issue_scatter(p)

    issue_idx(0, 0)
    wait_idx_make_gidx(0)
    issue_gather(0)
    issue_idx(1, 1)
    wait_gather(0)
    wait_idx_make_gidx(1)
    issue_gather(1)
    issue_scatter(0)

    def pair(g, carry):
        half(2 * g + 1, 1)
        half(2 * g + 2, 0)
        return carry

    NW = EPT // KW
    lax.fori_loop(0, (NW - 3) // 2, pair, 0)
    half(NW - 2, 1)
    wait_scatter(1)
    wait_gather(0)
    issue_scatter(0)
    wait_scatter(0)
    plsc.subcore_barrier()

    def wout(j, carry):
        r0 = s * 640 + j * 128
        pltpu.sync_copy(agg_sh.at[pl.ds(r0, 128)], agg_out.at[c, pl.ds(r0, 128)])
        return carry

    lax.fori_loop(0, 5, wout, 0)


RB = 2000


def _split_pad_stack(hn):
    a = jnp.pad(hn[:, :DH], ((0, 0), (0, DP - DH)))
    b = jnp.pad(hn[:, DH:], ((0, 0), (0, DP - DH)))
    return jnp.stack([a, b], axis=0)


def _tc_pre_body(x_ref, w_ref, b_ref, degp_ref, hn_ref, selfb_ref, inv_ref):
    deg = degp_ref[0] + degp_ref[1] + 1.0
    inv = lax.rsqrt(deg)
    h = jnp.dot(x_ref[...], w_ref[...], preferred_element_type=jnp.float32)
    hn_ref[...] = _split_pad_stack(h * inv)
    selfb_ref[...] = h * (inv * inv) + b_ref[...]
    inv_ref[...] = inv


def _tc_pre(x, w, bias, degp):
    return pl.pallas_call(
        _tc_pre_body,
        grid=(N // RB,),
        in_specs=[
            pl.BlockSpec((RB, D), lambda i: (i, 0)),
            pl.BlockSpec((D, D), lambda i: (0, 0)),
            pl.BlockSpec((1, D), lambda i: (0, 0)),
            pl.BlockSpec((NC, RB, 1), lambda i: (0, i, 0)),
        ],
        out_specs=[
            pl.BlockSpec((NC, RB, DP), lambda i: (0, i, 0)),
            pl.BlockSpec((RB, D), lambda i: (i, 0)),
            pl.BlockSpec((RB, 1), lambda i: (i, 0)),
        ],
        out_shape=[
            jax.ShapeDtypeStruct((NC, N, DP), jnp.float32),
            jax.ShapeDtypeStruct((N, D), jnp.float32),
            jax.ShapeDtypeStruct((N, 1), jnp.float32),
        ],
    )(x, w, bias, degp)


def _gcn_out(agg_ref, selfb_ref, inv_ref):
    aggcat = jnp.concatenate([agg_ref[0, :, :DH], agg_ref[1, :, :DH]], axis=1)
    return jnp.maximum(inv_ref[...] * aggcat + selfb_ref[...], 0.0)


def _tc_mid_body(agg_ref, selfb_ref, inv_ref, w_ref, b_ref, hn_ref, selfb2_ref):
    g = _gcn_out(agg_ref, selfb_ref, inv_ref)
    h = jnp.dot(g, w_ref[...], preferred_element_type=jnp.float32)
    inv = inv_ref[...]
    hn_ref[...] = _split_pad_stack(h * inv)
    selfb2_ref[...] = h * (inv * inv) + b_ref[...]


def _tc_mid(agg, selfb, inv, w, bias):
    return pl.pallas_call(
        _tc_mid_body,
        grid=(N // RB,),
        in_specs=[
            pl.BlockSpec((NC, RB, DP), lambda i: (0, i, 0)),
            pl.BlockSpec((RB, D), lambda i: (i, 0)),
            pl.BlockSpec((RB, 1), lambda i: (i, 0)),
            pl.BlockSpec((D, D), lambda i: (0, 0)),
            pl.BlockSpec((1, D), lambda i: (0, 0)),
        ],
        out_specs=[
            pl.BlockSpec((NC, RB, DP), lambda i: (0, i, 0)),
            pl.BlockSpec((RB, D), lambda i: (i, 0)),
        ],
        out_shape=[
            jax.ShapeDtypeStruct((NC, N, DP), jnp.float32),
            jax.ShapeDtypeStruct((N, D), jnp.float32),
        ],
    )(agg, selfb, inv, w, bias)


def _tc_final_body(agg_ref, selfb_ref, inv_ref, mol_ref, w1a_ref, w1b_ref,
                   b1_ref, w2_ref, b2_ref, w3_ref, b3_ref, w4_ref, b4_ref,
                   out_ref, acc_ref):
    i = pl.program_id(0)
    g = _gcn_out(agg_ref, selfb_ref, inv_ref)
    part = jnp.sum(g, axis=0, keepdims=True)

    @pl.when(i == 0)
    def _():
        acc_ref[...] = part

    @pl.when(i > 0)
    def _():
        acc_ref[...] = acc_ref[...] + part

    @pl.when(i == pl.num_programs(0) - 1)
    def _():
        pooled = acc_ref[...] * (1.0 / N)
        t = (jnp.dot(mol_ref[...], w1a_ref[...], preferred_element_type=jnp.float32)
             + jnp.dot(pooled, w1b_ref[...], preferred_element_type=jnp.float32)
             + b1_ref[...])
        t = jnp.maximum(t, 0.0)
        t = jnp.maximum(jnp.dot(t, w2_ref[...], preferred_element_type=jnp.float32) + b2_ref[...], 0.0)
        t = jnp.maximum(jnp.dot(t, w3_ref[...], preferred_element_type=jnp.float32) + b3_ref[...], 0.0)
        out_ref[...] = jnp.dot(t, w4_ref[...], preferred_element_type=jnp.float32) + b4_ref[...]


def _tc_final(agg, selfb, inv, mol_emb, w1a, w1b, b1, w2, b2, w3, b3, w4, b4):
    full = lambda shape: pl.BlockSpec(shape, lambda i: tuple(0 for _ in shape))
    h = w2.shape[0]
    o = w4.shape[1]
    return pl.pallas_call(
        _tc_final_body,
        grid=(N // RB,),
        in_specs=[
            pl.BlockSpec((NC, RB, DP), lambda i: (0, i, 0)),
            pl.BlockSpec((RB, D), lambda i: (i, 0)),
            pl.BlockSpec((RB, 1), lambda i: (i, 0)),
            full((B, D)),
            full((D, h)),
            full((D, h)),
            full((1, h)),
            full((h, h)),
            full((1, h)),
            full((h, h)),
            full((1, h)),
            full((h, o)),
            full((1, o)),
        ],
        out_specs=full((B, o)),
        out_shape=jax.ShapeDtypeStruct((B, o), jnp.float32),
        scratch_shapes=[pltpu.VMEM((1, D), jnp.float32)],
    )(agg, selfb, inv, mol_emb, w1a, w1b, b1, w2, b2, w3, b3, w4, b4)


def kernel(mol_emb, gene_ids, edge_index, emb_table, W_g1, b_g1, W_g2, b_g2,
           W1, b1, W2, b2, W3, b3, W4, b4):
    src = edge_index[0]
    dst = edge_index[1]
    degp = _sc_degree_kernel()(dst)
    degp_n = degp[:, :N].reshape(NC, N, 1)
    hn1, selfb1, inv = _tc_pre(emb_table, W_g1, b_g1.reshape(1, D), degp_n)
    agg1 = _sc_spmm_kernel()(hn1.reshape(NC * N, DP), src, dst)
    hn2, selfb2 = _tc_mid(agg1, selfb1, inv, W_g2, b_g2.reshape(1, D))
    agg2 = _sc_spmm_kernel()(hn2.reshape(NC * N, DP), src, dst)
    return _tc_final(agg2, selfb2, inv, mol_emb,
                     W1[:D], W1[D:], b1.reshape(1, -1),
                     W2, b2.reshape(1, -1), W3, b3.reshape(1, -1),
                     W4, b4.reshape(1, -1))
